# Initial kernel scaffold; baseline (speedup 1.0000x reference)
import functools

import jax
import jax.numpy as jnp
from jax import lax
from jax.experimental import pallas as pl
from jax.experimental.pallas import tpu as pltpu
from jax.experimental.pallas import tpu_sc as plsc

K0 = 7.5
K1 = 4.1
K2 = 19.09
K3 = 254.56
EPS = 1e-06

_P = 0.3275911
_A1 = 0.254829592
_A2 = -0.284496736
_A3 = 1.421413741
_A4 = -1.453152027
_A5 = 1.061405429

NC = 2
NS = 16
NW = NC * NS
L = 16

TBL = 128
CHUNK = 2000


@functools.lru_cache(maxsize=None)
def _build(n_edges, n_nodes):
    assert n_edges % (NW * CHUNK) == 0, (n_edges, NW * CHUNK)
    epw = n_edges // NW
    nchunk = epw // CHUNK
    assert nchunk % 2 == 0

    mesh = plsc.VectorSubcoreMesh(core_axis_name="c", subcore_axis_name="s")

    def body(z_hbm, ei_hbm, dist_hbm, rt_hbm, ct_hbm, en_hbm, out_hbm,
             z_v, rc_v, en_v, rt_v, ct_v,
             row_v, col_v, dist_v, out_v,
             in_sem0, in_sem1, out_sem0, out_sem1):
        wid = lax.axis_index("s") * NC + lax.axis_index("c")
        base = wid * epw
        in_sems = (in_sem0, in_sem1)
        out_sems = (out_sem0, out_sem1)

        pltpu.sync_copy(z_hbm, z_v)
        pltpu.sync_copy(rt_hbm, rt_v)
        pltpu.sync_copy(ct_hbm, ct_v)
        pltpu.sync_copy(en_hbm, en_v)
        for t in range(TBL // L):
            sl = pl.ds(t * L, L)
            rc_v[sl] = rt_v[sl] + ct_v[sl]

        def in_copies(g, b):
            off = pl.multiple_of(base + g * CHUNK, 8)
            return (
                pltpu.make_async_copy(ei_hbm.at[0, pl.ds(off, CHUNK)],
                                      row_v.at[b], in_sems[b]),
                pltpu.make_async_copy(ei_hbm.at[1, pl.ds(off, CHUNK)],
                                      col_v.at[b], in_sems[b]),
                pltpu.make_async_copy(dist_hbm.at[pl.ds(off, CHUNK)],
                                      dist_v.at[b], in_sems[b]),
            )

        def out_copy(g, b):
            off = pl.multiple_of(base + g * CHUNK, 8)
            return pltpu.make_async_copy(out_v.at[b],
                                         out_hbm.at[pl.ds(off, CHUNK)],
                                         out_sems[b])

        def compute(b):
            def vbody(v, carry):
                sl = pl.ds(pl.multiple_of(v * L, L), L)
                r16 = row_v[b, sl]
                c16 = col_v[b, sl]
                zi = plsc.load_gather(z_v, [r16])
                zj = plsc.load_gather(z_v, [c16])
                ri = plsc.load_gather(rc_v, [zi])
                rj = plsc.load_gather(rc_v, [zj])
                eni = plsc.load_gather(en_v, [zi])
                enj = plsc.load_gather(en_v, [zj])
                rcov = ri + rj
                d16 = dist_v[b, sl]
                u = jnp.abs(eni - enj) + K2
                den = (0.5 * K1) * jnp.exp(u * u * (-1.0 / K3))
                x = (-K0) * (d16 - rcov) / (rcov + EPS)
                a = jnp.abs(x)
                t = 1.0 / (1.0 + _P * a)
                poly = ((((_A5 * t + _A4) * t + _A3) * t + _A2) * t + _A1) * t
                e = 1.0 - poly * jnp.exp(-(a * a))
                erfx = jnp.where(x < 0.0, -e, e)
                out_v[b, sl] = den * (1.0 + erfx)
                return carry
            lax.fori_loop(0, CHUNK // L, vbody, 0)

        for c in in_copies(0, 0):
            c.start()
        for c in in_copies(1, 1):
            c.start()

        def outer(it, carry):
            for b in range(2):
                g = it * 2 + b
                for c in in_copies(g, b):
                    c.wait()

                @pl.when(g >= 2)
                def _():
                    out_copy(g - 2, b).wait()

                compute(b)
                out_copy(g, b).start()

                @pl.when(g + 2 < nchunk)
                def _():
                    for c in in_copies(g + 2, b):
                        c.start()
            return carry

        lax.fori_loop(0, nchunk // 2, outer, 0)

        out_copy(nchunk - 2, 0).wait()
        out_copy(nchunk - 1, 1).wait()

    return pl.kernel(
        body,
        out_type=jax.ShapeDtypeStruct((n_edges,), jnp.float32),
        mesh=mesh,
        scratch_types=[
            pltpu.VMEM((n_nodes,), jnp.int32),
            pltpu.VMEM((TBL,), jnp.float32),
            pltpu.VMEM((TBL,), jnp.float32),
            pltpu.VMEM((TBL,), jnp.float32),
            pltpu.VMEM((TBL,), jnp.float32),
            pltpu.VMEM((2, CHUNK), jnp.int32),
            pltpu.VMEM((2, CHUNK), jnp.int32),
            pltpu.VMEM((2, CHUNK), jnp.float32),
            pltpu.VMEM((2, CHUNK), jnp.float32),
            pltpu.SemaphoreType.DMA,
            pltpu.SemaphoreType.DMA,
            pltpu.SemaphoreType.DMA,
            pltpu.SemaphoreType.DMA,
        ],
    )


def _pad_table(tbl):
    flat = tbl.reshape(-1).astype(jnp.float32)
    return jnp.pad(flat, (0, TBL - flat.shape[0]))


def kernel(z, dist, edge_index, en_table, radius_table, corr_table):
    n_edges = dist.shape[0]
    n_nodes = z.shape[0]
    fn = _build(n_edges, n_nodes)
    out = fn(z.astype(jnp.int32),
             edge_index.astype(jnp.int32),
             dist.astype(jnp.float32),
             _pad_table(radius_table),
             _pad_table(corr_table),
             _pad_table(en_table))
    return out.reshape(n_edges, 1)


# speedup vs baseline: 357.9583x; 357.9583x over previous
import functools

import jax
import jax.numpy as jnp
from jax import lax
from jax.experimental import pallas as pl
from jax.experimental.pallas import tpu as pltpu
from jax.experimental.pallas import tpu_sc as plsc

K0 = 7.5
K1 = 4.1
K2 = 19.09
K3 = 254.56
EPS = 1e-06

_P = 0.3275911
_A1 = 0.254829592
_A2 = -0.284496736
_A3 = 1.421413741
_A4 = -1.453152027
_A5 = 1.061405429

NC = 2
NS = 16
NW = NC * NS
L = 16

TBL = 128
CHUNK = 2000


@functools.lru_cache(maxsize=None)
def _build(n_edges, n_nodes):
    assert n_edges % (NW * CHUNK) == 0, (n_edges, NW * CHUNK)
    epw = n_edges // NW
    nchunk = epw // CHUNK
    assert nchunk % 2 == 0

    mesh = plsc.VectorSubcoreMesh(core_axis_name="c", subcore_axis_name="s")

    def body(z_hbm, row_hbm, col_hbm, dist_hbm, rt_hbm, ct_hbm, en_hbm, out_hbm,
             z_v, rc_v, en_v, rt_v, ct_v,
             row_v0, row_v1, col_v0, col_v1, dist_v0, dist_v1, out_v0, out_v1,
             in_sem0, in_sem1, out_sem0, out_sem1):
        wid = lax.axis_index("s") * NC + lax.axis_index("c")
        base = wid * epw
        in_sems = (in_sem0, in_sem1)
        out_sems = (out_sem0, out_sem1)
        row_vs = (row_v0, row_v1)
        col_vs = (col_v0, col_v1)
        dist_vs = (dist_v0, dist_v1)
        out_vs = (out_v0, out_v1)

        pltpu.sync_copy(z_hbm, z_v)
        pltpu.sync_copy(rt_hbm, rt_v)
        pltpu.sync_copy(ct_hbm, ct_v)
        pltpu.sync_copy(en_hbm, en_v)
        for t in range(TBL // L):
            sl = pl.ds(t * L, L)
            rc_v[sl] = rt_v[sl] + ct_v[sl]

        def in_copies(g, b):
            off = pl.multiple_of(base + g * CHUNK, 8)
            return (
                pltpu.make_async_copy(row_hbm.at[pl.ds(off, CHUNK)],
                                      row_vs[b], in_sems[b]),
                pltpu.make_async_copy(col_hbm.at[pl.ds(off, CHUNK)],
                                      col_vs[b], in_sems[b]),
                pltpu.make_async_copy(dist_hbm.at[pl.ds(off, CHUNK)],
                                      dist_vs[b], in_sems[b]),
            )

        def out_copy(g, b):
            off = pl.multiple_of(base + g * CHUNK, 8)
            return pltpu.make_async_copy(out_vs[b],
                                         out_hbm.at[pl.ds(off, CHUNK)],
                                         out_sems[b])

        def compute(b):
            def vbody(v, carry):
                sl = pl.ds(pl.multiple_of(v * L, L), L)
                r16 = row_vs[b][sl]
                c16 = col_vs[b][sl]
                zi = plsc.load_gather(z_v, [r16])
                zj = plsc.load_gather(z_v, [c16])
                ri = plsc.load_gather(rc_v, [zi])
                rj = plsc.load_gather(rc_v, [zj])
                eni = plsc.load_gather(en_v, [zi])
                enj = plsc.load_gather(en_v, [zj])
                rcov = ri + rj
                d16 = dist_vs[b][sl]
                u = jnp.abs(eni - enj) + K2
                den = (0.5 * K1) * jnp.exp(u * u * (-1.0 / K3))
                x = (-K0) * (d16 - rcov) / (rcov + EPS)
                a = jnp.abs(x)
                t = 1.0 / (1.0 + _P * a)
                poly = ((((_A5 * t + _A4) * t + _A3) * t + _A2) * t + _A1) * t
                e = 1.0 - poly * jnp.exp(-(a * a))
                erfx = jnp.where(x < 0.0, -e, e)
                out_vs[b][sl] = den * (1.0 + erfx)
                return carry
            lax.fori_loop(0, CHUNK // L, vbody, 0)

        for c in in_copies(0, 0):
            c.start()
        for c in in_copies(1, 1):
            c.start()

        def outer(it, carry):
            for b in range(2):
                g = it * 2 + b
                for c in in_copies(g, b):
                    c.wait()

                @pl.when(g >= 2)
                def _():
                    out_copy(g - 2, b).wait()

                compute(b)
                out_copy(g, b).start()

                @pl.when(g + 2 < nchunk)
                def _():
                    for c in in_copies(g + 2, b):
                        c.start()
            return carry

        lax.fori_loop(0, nchunk // 2, outer, 0)

        out_copy(nchunk - 2, 0).wait()
        out_copy(nchunk - 1, 1).wait()

    return pl.kernel(
        body,
        out_type=jax.ShapeDtypeStruct((n_edges,), jnp.float32),
        mesh=mesh,
        compiler_params=pltpu.CompilerParams(needs_layout_passes=False),
        scratch_types=[
            pltpu.VMEM((n_nodes,), jnp.int32),
            pltpu.VMEM((TBL,), jnp.float32),
            pltpu.VMEM((TBL,), jnp.float32),
            pltpu.VMEM((TBL,), jnp.float32),
            pltpu.VMEM((TBL,), jnp.float32),
            pltpu.VMEM((CHUNK,), jnp.int32),
            pltpu.VMEM((CHUNK,), jnp.int32),
            pltpu.VMEM((CHUNK,), jnp.int32),
            pltpu.VMEM((CHUNK,), jnp.int32),
            pltpu.VMEM((CHUNK,), jnp.float32),
            pltpu.VMEM((CHUNK,), jnp.float32),
            pltpu.VMEM((CHUNK,), jnp.float32),
            pltpu.VMEM((CHUNK,), jnp.float32),
            pltpu.SemaphoreType.DMA,
            pltpu.SemaphoreType.DMA,
            pltpu.SemaphoreType.DMA,
            pltpu.SemaphoreType.DMA,
        ],
    )


def _pad_table(tbl):
    flat = tbl.reshape(-1).astype(jnp.float32)
    return jnp.pad(flat, (0, TBL - flat.shape[0]))


def kernel(z, dist, edge_index, en_table, radius_table, corr_table):
    n_edges = dist.shape[0]
    n_nodes = z.shape[0]
    fn = _build(n_edges, n_nodes)
    ei = edge_index.astype(jnp.int32)
    out = fn(z.astype(jnp.int32),
             ei[0],
             ei[1],
             dist.astype(jnp.float32),
             _pad_table(radius_table),
             _pad_table(corr_table),
             _pad_table(en_table))
    return out.reshape(n_edges, 1)


# speedup vs baseline: 993.7104x; 2.7761x over previous
import functools

import jax
import jax.numpy as jnp
from jax import lax
from jax.experimental import pallas as pl
from jax.experimental.pallas import tpu as pltpu
from jax.experimental.pallas import tpu_sc as plsc

K0 = 7.5
K1 = 4.1
K2 = 19.09
K3 = 254.56
EPS = 1e-06

_P = 0.3275911
_A1 = 0.254829592
_A2 = -0.284496736
_A3 = 1.421413741
_A4 = -1.453152027
_A5 = 1.061405429

NC = 2
NS = 16
NW = NC * NS
L = 16

TBL = 128
CHUNK = 2000


@functools.lru_cache(maxsize=None)
def _build(n_edges, n_nodes):
    assert n_edges % (NW * CHUNK) == 0, (n_edges, NW * CHUNK)
    epw = n_edges // NW
    nchunk = epw // CHUNK
    assert nchunk % 2 == 0

    mesh = plsc.VectorSubcoreMesh(core_axis_name="c", subcore_axis_name="s")

    def body(z_hbm, row_hbm, col_hbm, dist_hbm, rt_hbm, ct_hbm, en_hbm, out_hbm,
             z_v, rc_v, en_v, rt_v, ct_v,
             row_v0, row_v1, col_v0, col_v1, dist_v0, dist_v1, out_v0, out_v1,
             in_sem0, in_sem1, out_sem0, out_sem1):
        wid = lax.axis_index("s") * NC + lax.axis_index("c")
        base = wid * epw
        in_sems = (in_sem0, in_sem1)
        out_sems = (out_sem0, out_sem1)
        row_vs = (row_v0, row_v1)
        col_vs = (col_v0, col_v1)
        dist_vs = (dist_v0, dist_v1)
        out_vs = (out_v0, out_v1)

        pltpu.sync_copy(z_hbm, z_v)
        pltpu.sync_copy(rt_hbm, rt_v)
        pltpu.sync_copy(ct_hbm, ct_v)
        pltpu.sync_copy(en_hbm, en_v)
        for t in range(TBL // L):
            sl = pl.ds(t * L, L)
            rc_v[sl] = rt_v[sl] + ct_v[sl]

        def in_copies(g, b):
            off = pl.multiple_of(base + g * CHUNK, 8)
            return (
                pltpu.make_async_copy(row_hbm.at[pl.ds(off, CHUNK)],
                                      row_vs[b], in_sems[b]),
                pltpu.make_async_copy(col_hbm.at[pl.ds(off, CHUNK)],
                                      col_vs[b], in_sems[b]),
                pltpu.make_async_copy(dist_hbm.at[pl.ds(off, CHUNK)],
                                      dist_vs[b], in_sems[b]),
            )

        def out_copy(g, b):
            off = pl.multiple_of(base + g * CHUNK, 8)
            return pltpu.make_async_copy(out_vs[b],
                                         out_hbm.at[pl.ds(off, CHUNK)],
                                         out_sems[b])

        def compute(b):
            @plsc.parallel_loop(0, CHUNK // L, unroll=4)
            def vbody(v):
                sl = pl.ds(pl.multiple_of(v * L, L), L)
                r16 = row_vs[b][sl]
                c16 = col_vs[b][sl]
                zi = plsc.load_gather(z_v, [r16])
                zj = plsc.load_gather(z_v, [c16])
                ri = plsc.load_gather(rc_v, [zi])
                rj = plsc.load_gather(rc_v, [zj])
                eni = plsc.load_gather(en_v, [zi])
                enj = plsc.load_gather(en_v, [zj])
                rcov = ri + rj
                d16 = dist_vs[b][sl]
                u = jnp.abs(eni - enj) + K2
                den = (0.5 * K1) * jnp.exp(u * u * (-1.0 / K3))
                x = (-K0) * (d16 - rcov) / (rcov + EPS)
                a = jnp.abs(x)
                t = 1.0 / (1.0 + _P * a)
                poly = ((((_A5 * t + _A4) * t + _A3) * t + _A2) * t + _A1) * t
                e = 1.0 - poly * jnp.exp(-(a * a))
                erfx = jnp.where(x < 0.0, -e, e)
                out_vs[b][sl] = den * (1.0 + erfx)

        for c in in_copies(0, 0):
            c.start()
        for c in in_copies(1, 1):
            c.start()

        def outer(it, carry):
            for b in range(2):
                g = it * 2 + b
                for c in in_copies(g, b):
                    c.wait()

                @pl.when(g >= 2)
                def _():
                    out_copy(g - 2, b).wait()

                compute(b)
                out_copy(g, b).start()

                @pl.when(g + 2 < nchunk)
                def _():
                    for c in in_copies(g + 2, b):
                        c.start()
            return carry

        lax.fori_loop(0, nchunk // 2, outer, 0)

        out_copy(nchunk - 2, 0).wait()
        out_copy(nchunk - 1, 1).wait()

    return pl.kernel(
        body,
        out_type=jax.ShapeDtypeStruct((n_edges,), jnp.float32),
        mesh=mesh,
        compiler_params=pltpu.CompilerParams(needs_layout_passes=False),
        scratch_types=[
            pltpu.VMEM((n_nodes,), jnp.int32),
            pltpu.VMEM((TBL,), jnp.float32),
            pltpu.VMEM((TBL,), jnp.float32),
            pltpu.VMEM((TBL,), jnp.float32),
            pltpu.VMEM((TBL,), jnp.float32),
            pltpu.VMEM((CHUNK,), jnp.int32),
            pltpu.VMEM((CHUNK,), jnp.int32),
            pltpu.VMEM((CHUNK,), jnp.int32),
            pltpu.VMEM((CHUNK,), jnp.int32),
            pltpu.VMEM((CHUNK,), jnp.float32),
            pltpu.VMEM((CHUNK,), jnp.float32),
            pltpu.VMEM((CHUNK,), jnp.float32),
            pltpu.VMEM((CHUNK,), jnp.float32),
            pltpu.SemaphoreType.DMA,
            pltpu.SemaphoreType.DMA,
            pltpu.SemaphoreType.DMA,
            pltpu.SemaphoreType.DMA,
        ],
    )


def _pad_table(tbl):
    flat = tbl.reshape(-1).astype(jnp.float32)
    return jnp.pad(flat, (0, TBL - flat.shape[0]))


def kernel(z, dist, edge_index, en_table, radius_table, corr_table):
    n_edges = dist.shape[0]
    n_nodes = z.shape[0]
    fn = _build(n_edges, n_nodes)
    ei = edge_index.astype(jnp.int32)
    out = fn(z.astype(jnp.int32),
             ei[0],
             ei[1],
             dist.astype(jnp.float32),
             _pad_table(radius_table),
             _pad_table(corr_table),
             _pad_table(en_table))
    return out.reshape(n_edges, 1)


# speedup vs baseline: 1093.2929x; 1.1002x over previous
import functools

import jax
import jax.numpy as jnp
from jax import lax
from jax.experimental import pallas as pl
from jax.experimental.pallas import tpu as pltpu
from jax.experimental.pallas import tpu_sc as plsc

K0 = 7.5
K1 = 4.1
K2 = 19.09
K3 = 254.56
EPS = 1e-06

_EA = -2.0 * 1.12967583
_EB = -2.0 * 0.0997927

NC = 2
NS = 16
NW = NC * NS
L = 16

TBL = 128
CHUNK = 2000


@functools.lru_cache(maxsize=None)
def _build(n_edges, n_nodes):
    assert n_edges % (NW * CHUNK) == 0, (n_edges, NW * CHUNK)
    epw = n_edges // NW
    nchunk = epw // CHUNK
    assert nchunk % 2 == 0

    mesh = plsc.VectorSubcoreMesh(core_axis_name="c", subcore_axis_name="s")

    def body(z_hbm, row_hbm, col_hbm, dist_hbm, rt_hbm, ct_hbm, en_hbm, out_hbm,
             z_v, rc_v, en_v, rt_v, ct_v,
             row_v0, row_v1, col_v0, col_v1, dist_v0, dist_v1, out_v0, out_v1,
             in_sem0, in_sem1, out_sem0, out_sem1):
        wid = lax.axis_index("s") * NC + lax.axis_index("c")
        base = wid * epw
        in_sems = (in_sem0, in_sem1)
        out_sems = (out_sem0, out_sem1)
        row_vs = (row_v0, row_v1)
        col_vs = (col_v0, col_v1)
        dist_vs = (dist_v0, dist_v1)
        out_vs = (out_v0, out_v1)

        pltpu.sync_copy(z_hbm, z_v)
        pltpu.sync_copy(rt_hbm, rt_v)
        pltpu.sync_copy(ct_hbm, ct_v)
        pltpu.sync_copy(en_hbm, en_v)
        for t in range(TBL // L):
            sl = pl.ds(t * L, L)
            rc_v[sl] = rt_v[sl] + ct_v[sl]

        def in_copies(g, b):
            off = pl.multiple_of(base + g * CHUNK, 8)
            return (
                pltpu.make_async_copy(row_hbm.at[pl.ds(off, CHUNK)],
                                      row_vs[b], in_sems[b]),
                pltpu.make_async_copy(col_hbm.at[pl.ds(off, CHUNK)],
                                      col_vs[b], in_sems[b]),
                pltpu.make_async_copy(dist_hbm.at[pl.ds(off, CHUNK)],
                                      dist_vs[b], in_sems[b]),
            )

        def out_copy(g, b):
            off = pl.multiple_of(base + g * CHUNK, 8)
            return pltpu.make_async_copy(out_vs[b],
                                         out_hbm.at[pl.ds(off, CHUNK)],
                                         out_sems[b])

        def compute(b):
            @plsc.parallel_loop(0, CHUNK // L, unroll=4)
            def vbody(v):
                sl = pl.ds(pl.multiple_of(v * L, L), L)
                r16 = row_vs[b][sl]
                c16 = col_vs[b][sl]
                zi = plsc.load_gather(z_v, [r16])
                zj = plsc.load_gather(z_v, [c16])
                ri = plsc.load_gather(rc_v, [zi])
                rj = plsc.load_gather(rc_v, [zj])
                eni = plsc.load_gather(en_v, [zi])
                enj = plsc.load_gather(en_v, [zj])
                rcov = ri + rj
                d16 = dist_vs[b][sl]
                u = jnp.abs(eni - enj) + K2
                den = (0.5 * K1) * jnp.exp(u * u * (-1.0 / K3))
                x = (-K0) * (d16 - rcov) / (rcov + EPS)
                w = x * (_EA + _EB * (x * x))
                out_vs[b][sl] = (den + den) / (1.0 + jnp.exp(w))

        for c in in_copies(0, 0):
            c.start()
        for c in in_copies(1, 1):
            c.start()

        def outer(it, carry):
            for b in range(2):
                g = it * 2 + b
                for c in in_copies(g, b):
                    c.wait()

                @pl.when(g >= 2)
                def _():
                    out_copy(g - 2, b).wait()

                compute(b)
                out_copy(g, b).start()

                @pl.when(g + 2 < nchunk)
                def _():
                    for c in in_copies(g + 2, b):
                        c.start()
            return carry

        lax.fori_loop(0, nchunk // 2, outer, 0)

        out_copy(nchunk - 2, 0).wait()
        out_copy(nchunk - 1, 1).wait()

    return pl.kernel(
        body,
        out_type=jax.ShapeDtypeStruct((n_edges,), jnp.float32),
        mesh=mesh,
        compiler_params=pltpu.CompilerParams(needs_layout_passes=False),
        scratch_types=[
            pltpu.VMEM((n_nodes,), jnp.int32),
            pltpu.VMEM((TBL,), jnp.float32),
            pltpu.VMEM((TBL,), jnp.float32),
            pltpu.VMEM((TBL,), jnp.float32),
            pltpu.VMEM((TBL,), jnp.float32),
            pltpu.VMEM((CHUNK,), jnp.int32),
            pltpu.VMEM((CHUNK,), jnp.int32),
            pltpu.VMEM((CHUNK,), jnp.int32),
            pltpu.VMEM((CHUNK,), jnp.int32),
            pltpu.VMEM((CHUNK,), jnp.float32),
            pltpu.VMEM((CHUNK,), jnp.float32),
            pltpu.VMEM((CHUNK,), jnp.float32),
            pltpu.VMEM((CHUNK,), jnp.float32),
            pltpu.SemaphoreType.DMA,
            pltpu.SemaphoreType.DMA,
            pltpu.SemaphoreType.DMA,
            pltpu.SemaphoreType.DMA,
        ],
    )


def _pad_table(tbl):
    flat = tbl.reshape(-1).astype(jnp.float32)
    return jnp.pad(flat, (0, TBL - flat.shape[0]))


def kernel(z, dist, edge_index, en_table, radius_table, corr_table):
    n_edges = dist.shape[0]
    n_nodes = z.shape[0]
    fn = _build(n_edges, n_nodes)
    ei = edge_index.astype(jnp.int32)
    out = fn(z.astype(jnp.int32),
             ei[0],
             ei[1],
             dist.astype(jnp.float32),
             _pad_table(radius_table),
             _pad_table(corr_table),
             _pad_table(en_table))
    return out.reshape(n_edges, 1)


# speedup vs baseline: 1193.5533x; 1.0917x over previous
import functools

import jax
import jax.numpy as jnp
from jax import lax
from jax.experimental import pallas as pl
from jax.experimental.pallas import tpu as pltpu
from jax.experimental.pallas import tpu_sc as plsc

K0 = 7.5
K1 = 4.1
K2 = 19.09
K3 = 254.56
EPS = 1e-06

_EA = -2.0 * 1.12967583
_EB = -2.0 * 0.0997927

NC = 2
NS = 16
NW = NC * NS
L = 16

TBL = 128
CHUNK = 2000


@functools.lru_cache(maxsize=None)
def _build(n_edges, n_nodes):
    assert n_edges % (NW * CHUNK) == 0, (n_edges, NW * CHUNK)
    epw = n_edges // NW
    nchunk = epw // CHUNK
    assert nchunk % 2 == 0

    mesh = plsc.VectorSubcoreMesh(core_axis_name="c", subcore_axis_name="s")

    def body(z_hbm, ei_hbm, dist_hbm, rt_hbm, ct_hbm, en_hbm, out_hbm,
             z_v, rc_v, en_v, rt_v, ct_v,
             row_v0, row_v1, col_v0, col_v1, dist_v0, dist_v1, out_v0, out_v1,
             in_sem0, in_sem1, out_sem0, out_sem1):
        wid = lax.axis_index("s") * NC + lax.axis_index("c")
        base = wid * epw
        in_sems = (in_sem0, in_sem1)
        out_sems = (out_sem0, out_sem1)
        row_vs = (row_v0, row_v1)
        col_vs = (col_v0, col_v1)
        dist_vs = (dist_v0, dist_v1)
        out_vs = (out_v0, out_v1)

        pltpu.sync_copy(z_hbm, z_v)
        pltpu.sync_copy(rt_hbm, rt_v)
        pltpu.sync_copy(ct_hbm, ct_v)
        pltpu.sync_copy(en_hbm, en_v)
        for t in range(TBL // L):
            sl = pl.ds(t * L, L)
            rc_v[sl] = rt_v[sl] + ct_v[sl]

        def in_copies(g, b):
            off = pl.multiple_of(base + g * CHUNK, 8)
            return (
                pltpu.make_async_copy(ei_hbm.at[pl.ds(off, CHUNK)],
                                      row_vs[b], in_sems[b]),
                pltpu.make_async_copy(ei_hbm.at[pl.ds(off + n_edges, CHUNK)],
                                      col_vs[b], in_sems[b]),
                pltpu.make_async_copy(dist_hbm.at[pl.ds(off, CHUNK)],
                                      dist_vs[b], in_sems[b]),
            )

        def out_copy(g, b):
            off = pl.multiple_of(base + g * CHUNK, 8)
            return pltpu.make_async_copy(out_vs[b],
                                         out_hbm.at[pl.ds(off, CHUNK)],
                                         out_sems[b])

        def compute(b):
            @plsc.parallel_loop(0, CHUNK // L, unroll=4)
            def vbody(v):
                sl = pl.ds(pl.multiple_of(v * L, L), L)
                r16 = row_vs[b][sl]
                c16 = col_vs[b][sl]
                zi = plsc.load_gather(z_v, [r16])
                zj = plsc.load_gather(z_v, [c16])
                ri = plsc.load_gather(rc_v, [zi])
                rj = plsc.load_gather(rc_v, [zj])
                eni = plsc.load_gather(en_v, [zi])
                enj = plsc.load_gather(en_v, [zj])
                rcov = ri + rj
                d16 = dist_vs[b][sl]
                u = jnp.abs(eni - enj) + K2
                den = (0.5 * K1) * jnp.exp(u * u * (-1.0 / K3))
                x = (-K0) * (d16 - rcov) / (rcov + EPS)
                w = x * (_EA + _EB * (x * x))
                out_vs[b][sl] = (den + den) / (1.0 + jnp.exp(w))

        for c in in_copies(0, 0):
            c.start()
        for c in in_copies(1, 1):
            c.start()

        def outer(it, carry):
            for b in range(2):
                g = it * 2 + b
                for c in in_copies(g, b):
                    c.wait()

                @pl.when(g >= 2)
                def _():
                    out_copy(g - 2, b).wait()

                compute(b)
                out_copy(g, b).start()

                @pl.when(g + 2 < nchunk)
                def _():
                    for c in in_copies(g + 2, b):
                        c.start()
            return carry

        lax.fori_loop(0, nchunk // 2, outer, 0)

        out_copy(nchunk - 2, 0).wait()
        out_copy(nchunk - 1, 1).wait()

    return pl.kernel(
        body,
        out_type=jax.ShapeDtypeStruct((n_edges,), jnp.float32),
        mesh=mesh,
        compiler_params=pltpu.CompilerParams(needs_layout_passes=False),
        scratch_types=[
            pltpu.VMEM((n_nodes,), jnp.int32),
            pltpu.VMEM((TBL,), jnp.float32),
            pltpu.VMEM((TBL,), jnp.float32),
            pltpu.VMEM((TBL,), jnp.float32),
            pltpu.VMEM((TBL,), jnp.float32),
            pltpu.VMEM((CHUNK,), jnp.int32),
            pltpu.VMEM((CHUNK,), jnp.int32),
            pltpu.VMEM((CHUNK,), jnp.int32),
            pltpu.VMEM((CHUNK,), jnp.int32),
            pltpu.VMEM((CHUNK,), jnp.float32),
            pltpu.VMEM((CHUNK,), jnp.float32),
            pltpu.VMEM((CHUNK,), jnp.float32),
            pltpu.VMEM((CHUNK,), jnp.float32),
            pltpu.SemaphoreType.DMA,
            pltpu.SemaphoreType.DMA,
            pltpu.SemaphoreType.DMA,
            pltpu.SemaphoreType.DMA,
        ],
    )


def _pad_table(tbl):
    flat = tbl.reshape(-1).astype(jnp.float32)
    return jnp.pad(flat, (0, TBL - flat.shape[0]))


def kernel(z, dist, edge_index, en_table, radius_table, corr_table):
    n_edges = dist.shape[0]
    n_nodes = z.shape[0]
    fn = _build(n_edges, n_nodes)
    out = fn(z.astype(jnp.int32),
             edge_index.astype(jnp.int32).reshape(-1),
             dist.astype(jnp.float32),
             _pad_table(radius_table),
             _pad_table(corr_table),
             _pad_table(en_table))
    return out.reshape(n_edges, 1)


# speedup vs baseline: 1556.1965x; 1.3038x over previous
import functools

import jax
import jax.numpy as jnp
from jax import lax
from jax.experimental import pallas as pl
from jax.experimental.pallas import tpu as pltpu
from jax.experimental.pallas import tpu_sc as plsc

K0 = 7.5
K1 = 4.1
K2 = 19.09
K3 = 254.56
EPS = 1e-06

_EA = -2.0 * 1.12967583
_EB = -2.0 * 0.0997927

NC = 2
NS = 16
NW = NC * NS
L = 16

TBL = 128
CHUNK = 2560


@functools.lru_cache(maxsize=None)
def _build(n_edges, n_nodes):
    assert n_edges % CHUNK == 0, (n_edges, CHUNK)
    nch = n_edges // CHUNK
    assert nch >= NW
    nl = -(-nch // NW)
    if nl % 2:
        nl += 1
    mesh = plsc.VectorSubcoreMesh(core_axis_name="c", subcore_axis_name="s")

    def body(z_hbm, ei_hbm, dist_hbm, rt_hbm, ct_hbm, en_hbm, out_hbm,
             z_v, rc_v, en_v, rt_v, ct_v,
             ei_v0, ei_v1, dist_v0, dist_v1, out_v0, out_v1,
             in_sem0, in_sem1, out_sem0, out_sem1):
        wid = lax.axis_index("s") * NC + lax.axis_index("c")
        in_sems = (in_sem0, in_sem1)
        out_sems = (out_sem0, out_sem1)
        ei_vs = (ei_v0, ei_v1)
        dist_vs = (dist_v0, dist_v1)
        out_vs = (out_v0, out_v1)

        pltpu.sync_copy(z_hbm, z_v)
        pltpu.sync_copy(rt_hbm, rt_v)
        pltpu.sync_copy(ct_hbm, ct_v)
        pltpu.sync_copy(en_hbm, en_v)
        for t in range(TBL // L):
            sl = pl.ds(t * L, L)
            rc_v[sl] = rt_v[sl] + ct_v[sl]

        def glob(l):
            g = l * NW + wid
            return jnp.where(g < nch, g, wid)

        def in_copies(l, b):
            off = pl.multiple_of(glob(l) * CHUNK, 128)
            return (
                pltpu.make_async_copy(ei_hbm.at[:, pl.ds(off, CHUNK)],
                                      ei_vs[b], in_sems[b]),
                pltpu.make_async_copy(dist_hbm.at[pl.ds(off, CHUNK)],
                                      dist_vs[b], in_sems[b]),
            )

        def out_copy(l, b):
            off = pl.multiple_of(glob(l) * CHUNK, 128)
            return pltpu.make_async_copy(out_vs[b],
                                         out_hbm.at[pl.ds(off, CHUNK)],
                                         out_sems[b])

        def compute(b):
            @plsc.parallel_loop(0, CHUNK // L, unroll=4)
            def vbody(v):
                sl = pl.ds(pl.multiple_of(v * L, L), L)
                r16 = ei_vs[b][0, sl]
                c16 = ei_vs[b][1, sl]
                zi = plsc.load_gather(z_v, [r16])
                zj = plsc.load_gather(z_v, [c16])
                ri = plsc.load_gather(rc_v, [zi])
                rj = plsc.load_gather(rc_v, [zj])
                eni = plsc.load_gather(en_v, [zi])
                enj = plsc.load_gather(en_v, [zj])
                rcov = ri + rj
                d16 = dist_vs[b][sl]
                u = jnp.abs(eni - enj) + K2
                den = (0.5 * K1) * jnp.exp(u * u * (-1.0 / K3))
                x = (-K0) * (d16 - rcov) / (rcov + EPS)
                w = x * (_EA + _EB * (x * x))
                out_vs[b][sl] = (den + den) / (1.0 + jnp.exp(w))

        for c in in_copies(0, 0):
            c.start()
        for c in in_copies(1, 1):
            c.start()

        def outer(it, carry):
            for b in range(2):
                l = it * 2 + b
                for c in in_copies(l, b):
                    c.wait()

                @pl.when(l >= 2)
                def _():
                    out_copy(l, b).wait()

                compute(b)
                out_copy(l, b).start()

                @pl.when(l + 2 < nl)
                def _():
                    for c in in_copies(l + 2, b):
                        c.start()
            return carry

        lax.fori_loop(0, nl // 2, outer, 0)

        out_copy(nl - 2, 0).wait()
        out_copy(nl - 1, 1).wait()

    return pl.kernel(
        body,
        out_type=jax.ShapeDtypeStruct((n_edges,), jnp.float32),
        mesh=mesh,
        compiler_params=pltpu.CompilerParams(needs_layout_passes=False),
        scratch_types=[
            pltpu.VMEM((n_nodes,), jnp.int32),
            pltpu.VMEM((TBL,), jnp.float32),
            pltpu.VMEM((TBL,), jnp.float32),
            pltpu.VMEM((TBL,), jnp.float32),
            pltpu.VMEM((TBL,), jnp.float32),
            pltpu.VMEM((2, CHUNK), jnp.int32),
            pltpu.VMEM((2, CHUNK), jnp.int32),
            pltpu.VMEM((CHUNK,), jnp.float32),
            pltpu.VMEM((CHUNK,), jnp.float32),
            pltpu.VMEM((CHUNK,), jnp.float32),
            pltpu.VMEM((CHUNK,), jnp.float32),
            pltpu.SemaphoreType.DMA,
            pltpu.SemaphoreType.DMA,
            pltpu.SemaphoreType.DMA,
            pltpu.SemaphoreType.DMA,
        ],
    )


def _pad_table(tbl):
    flat = tbl.reshape(-1).astype(jnp.float32)
    return jnp.pad(flat, (0, TBL - flat.shape[0]))


def kernel(z, dist, edge_index, en_table, radius_table, corr_table):
    n_edges = dist.shape[0]
    n_nodes = z.shape[0]
    fn = _build(n_edges, n_nodes)
    out = fn(z.astype(jnp.int32),
             edge_index.astype(jnp.int32),
             dist.astype(jnp.float32),
             _pad_table(radius_table),
             _pad_table(corr_table),
             _pad_table(en_table))
    return out.reshape(n_edges, 1)
